# Initial kernel scaffold; baseline (speedup 1.0000x reference)
import jax
import jax.numpy as jnp
from jax.experimental import pallas as pl


def kernel(loc_pred, conf_pred, refined_anchors, ignore_flags_refined_anchor, targets):
    raise NotImplementedError("write your pallas kernel here")


# speedup vs baseline: 6.6199x; 6.6199x over previous
import functools

import jax
import jax.numpy as jnp
from jax import lax
from jax.experimental import pallas as pl
from jax.experimental.pallas import tpu as pltpu
from jax.experimental.pallas import tpu_sc as plsc

_NUM = 8
_NA = 5000
_NG = 16
_PAD_NA = 5120
_CHUNK = 1280
_VPT = _CHUNK // 16
_LN2 = 0.6931471805599453


def _vlog(x):
    bits = lax.bitcast_convert_type(x, jnp.int32)
    e = lax.shift_right_logical(bits, 23) - 127
    m = lax.bitcast_convert_type(
        jnp.bitwise_or(jnp.bitwise_and(bits, 0x007FFFFF), 0x3F800000),
        jnp.float32)
    big = m > 1.4142135623730951
    m = jnp.where(big, m * 0.5, m)
    e = jnp.where(big, e + 1, e)
    t = (m - 1.0) / (m + 1.0)
    t2 = t * t
    p = 1.0 + t2 * (1.0 / 3.0 + t2 * (1.0 / 5.0 + t2 * (1.0 / 7.0 + t2 * (1.0 / 9.0))))
    return e.astype(jnp.float32) * _LN2 + 2.0 * t * p


def _sc_body(ac_h, aw_h, l0_h, l1_h, ign_h, gs_h, ge_h, out_h,
             ac_v, aw_v, l0_v, l1_v, ign_v, gs_v, ge_v,
             pos_v, bidx_v, gmax_v, gidx_v,
             stage_v, tmp_v, mc_v, mw_v, res_v, sh):
    c = lax.axis_index("c")
    s = lax.axis_index("s")
    batch = c * 4 + s // 4
    chunk = s % 4
    base = batch * _PAD_NA + chunk * _CHUNK
    anchor0 = chunk * _CHUNK

    pltpu.sync_copy(ac_h.at[pl.ds(base, _CHUNK)], ac_v)
    pltpu.sync_copy(aw_h.at[pl.ds(base, _CHUNK)], aw_v)
    pltpu.sync_copy(l0_h.at[pl.ds(base, _CHUNK)], l0_v)
    pltpu.sync_copy(l1_h.at[pl.ds(base, _CHUNK)], l1_v)
    pltpu.sync_copy(ign_h.at[pl.ds(base, _CHUNK)], ign_v)
    pltpu.sync_copy(gs_h.at[pl.ds(batch * _NG, _NG)], gs_v)
    pltpu.sync_copy(ge_h.at[pl.ds(batch * _NG, _NG)], ge_v)

    lane = lax.iota(jnp.int32, 16)
    gs_vec = gs_v[...]
    ge_vec = ge_v[...]
    gs_s = [gs_vec[g] for g in range(_NG)]
    ge_s = [ge_vec[g] for g in range(_NG)]
    glen_s = [ge_s[g] - gs_s[g] for g in range(_NG)]

    for g in range(_NG):
        gmax_v[g] = jnp.full((16,), -1.0, jnp.float32)
        gidx_v[g] = jnp.zeros((16,), jnp.int32)

    def p1(i, _):
        sl = pl.ds(i * 16, 16)
        ac = ac_v[sl]
        aw = aw_v[sl]
        a_s = ac - aw / 2.0
        a_e = ac + aw / 2.0
        alen = a_e - a_s
        aidx = anchor0 + i * 16 + lane
        bov = jnp.full((16,), -1.0, jnp.float32)
        bidx = jnp.zeros((16,), jnp.int32)
        for g in range(_NG):
            inter = jnp.minimum(ge_s[g], a_e) - jnp.maximum(gs_s[g], a_s)
            inter = jnp.maximum(inter, 0.0)
            union = jnp.maximum(glen_s[g] + alen - inter, 1e-10)
            iou = inter / union
            upd = iou > bov
            bov = jnp.where(upd, iou, bov)
            bidx = jnp.where(upd, g, bidx)
            gm = gmax_v[g]
            gu = iou > gm
            gmax_v[g] = jnp.where(gu, iou, gm)
            gidx_v[g] = jnp.where(gu, aidx, gidx_v[g])
        pos_v[sl] = (bov >= 0.5).astype(jnp.int32)
        bidx_v[sl] = bidx
        return 0

    lax.fori_loop(0, _VPT, p1, 0)

    pk_i = jnp.zeros((16,), jnp.float32)
    pk_x = jnp.zeros((16,), jnp.int32)
    for g in range(_NG):
        row = gmax_v[g]
        m = jnp.max(row)
        cand = jnp.where(row == m, gidx_v[g], jnp.int32(2 ** 30))
        mi = jnp.min(cand)
        pk_i = jnp.where(lane == g, m, pk_i)
        pk_x = jnp.where(lane == g, mi, pk_x)

    stage_v[pl.ds(0, 16)] = pk_i
    stage_v[pl.ds(16, 16)] = lax.bitcast_convert_type(pk_x, jnp.float32)
    pltpu.sync_copy(stage_v, sh.at[pl.ds(s * 32, 32)])
    plsc.subcore_barrier()
    s0 = (s // 4) * 4
    pltpu.sync_copy(sh.at[pl.ds(s0 * 32, 32)], tmp_v)
    cur_i = tmp_v[pl.ds(0, 16)]
    cur_x = lax.bitcast_convert_type(tmp_v[pl.ds(16, 16)], jnp.int32)
    for cc in range(1, 4):
        pltpu.sync_copy(sh.at[pl.ds((s0 + cc) * 32, 32)], tmp_v)
        vi = tmp_v[pl.ds(0, 16)]
        vx = lax.bitcast_convert_type(tmp_v[pl.ds(16, 16)], jnp.int32)
        u = vi > cur_i
        cur_i = jnp.where(u, vi, cur_i)
        cur_x = jnp.where(u, vx, cur_x)
    plsc.subcore_barrier()

    ones_i = jnp.ones((16,), jnp.int32)
    for g in range(_NG):
        loc = cur_x[g] - anchor0
        inr = jnp.logical_and(loc >= 0, loc < _CHUNK)
        li = jnp.clip(loc, 0, _CHUNK - 1)
        idxv = jnp.broadcast_to(li, (16,))
        mask = jnp.logical_and(lane == g, inr)
        plsc.store_scatter(pos_v, [idxv], ones_i, mask=mask)
        plsc.store_scatter(bidx_v, [idxv], jnp.full((16,), g, jnp.int32), mask=mask)

    mc_v[...] = (gs_vec + ge_vec) / 2.0
    mw_v[...] = ge_vec - gs_vec

    def p2(i, carry):
        ls, cs = carry
        sl = pl.ds(i * 16, 16)
        posr = pos_v[sl]
        bidx = bidx_v[sl]
        ign = ign_v[sl]
        ac = ac_v[sl]
        aw = aw_v[sl]
        l0 = l0_v[sl]
        l1 = l1_v[sl]
        p = jnp.logical_and(posr > 0, ign == 0)
        mc = plsc.load_gather(mc_v, [bidx])
        mw = plsc.load_gather(mw_v, [bidx])
        lc = (mc - ac) / (0.1 * aw)
        r = jnp.maximum(mw / aw, 1e-10)
        lw = _vlog(r) / 0.2
        d0 = l0 - lc
        d1 = l1 - lw
        a0 = jnp.abs(d0)
        a1 = jnp.abs(d1)
        s0 = jnp.where(a0 < 1.0, 0.5 * a0 * a0, a0 - 0.5)
        s1 = jnp.where(a1 < 1.0, 0.5 * a1 * a1, a1 - 0.5)
        ls = ls + jnp.where(p, s0 + s1, 0.0)
        cs = cs + jnp.where(p, 1.0, 0.0)
        return ls, cs

    zero16 = jnp.zeros((16,), jnp.float32)
    ls, cs = lax.fori_loop(0, _VPT, p2, (zero16, zero16))
    lsum = jnp.sum(ls)
    csum = jnp.sum(cs)
    stage_v[pl.ds(0, 16)] = jnp.where(lane == 0, lsum, jnp.where(lane == 1, csum, 0.0))
    pltpu.sync_copy(stage_v, sh.at[pl.ds(s * 32, 32)])
    plsc.subcore_barrier()

    @pl.when(s == 0)
    def _():
        acc = jnp.zeros((16,), jnp.float32)
        for k in range(16):
            pltpu.sync_copy(sh.at[pl.ds(k * 32, 32)], tmp_v)
            acc = acc + tmp_v[pl.ds(0, 16)]
        res_v[...] = acc
        pltpu.sync_copy(res_v, out_h.at[pl.ds(c * 16, 16)])


@functools.partial(
    pl.kernel,
    mesh=plsc.VectorSubcoreMesh(core_axis_name="c", subcore_axis_name="s"),
    out_type=jax.ShapeDtypeStruct((32,), jnp.float32),
    compiler_params=pltpu.CompilerParams(needs_layout_passes=False),
    scratch_types=[
        pltpu.VMEM((_CHUNK,), jnp.float32),
        pltpu.VMEM((_CHUNK,), jnp.float32),
        pltpu.VMEM((_CHUNK,), jnp.float32),
        pltpu.VMEM((_CHUNK,), jnp.float32),
        pltpu.VMEM((_CHUNK,), jnp.int32),
        pltpu.VMEM((_NG,), jnp.float32),
        pltpu.VMEM((_NG,), jnp.float32),
        pltpu.VMEM((_CHUNK,), jnp.int32),
        pltpu.VMEM((_CHUNK,), jnp.int32),
        pltpu.VMEM((_NG, 16), jnp.float32),
        pltpu.VMEM((_NG, 16), jnp.int32),
        pltpu.VMEM((32,), jnp.float32),
        pltpu.VMEM((32,), jnp.float32),
        pltpu.VMEM((_NG,), jnp.float32),
        pltpu.VMEM((_NG,), jnp.float32),
        pltpu.VMEM((16,), jnp.float32),
        pltpu.VMEM_SHARED((512,), jnp.float32),
    ],
)
def _fgd_sc(*refs):
    _sc_body(*refs)


def kernel(loc_pred, conf_pred, refined_anchors, ignore_flags_refined_anchor, targets):
    del conf_pred
    pad = _PAD_NA - _NA
    ac = jnp.pad(refined_anchors[..., 0], ((0, 0), (0, pad)), constant_values=-10.0)
    aw = jnp.pad(refined_anchors[..., 1], ((0, 0), (0, pad)), constant_values=1.0)
    l0 = jnp.pad(loc_pred[..., 0], ((0, 0), (0, pad)))
    l1 = jnp.pad(loc_pred[..., 1], ((0, 0), (0, pad)))
    ign = jnp.pad(ignore_flags_refined_anchor, ((0, 0), (0, pad)), constant_values=1)
    out = _fgd_sc(ac.reshape(-1), aw.reshape(-1), l0.reshape(-1), l1.reshape(-1),
                  ign.reshape(-1),
                  targets[:, :, 0].reshape(-1), targets[:, :, 1].reshape(-1))
    return (out[0] + out[16]) / (out[1] + out[17])
